# baseline (speedup 1.0000x reference)
import functools

import jax
import jax.numpy as jnp
from jax import lax
from jax.experimental import pallas as pl
from jax.experimental.pallas import tpu as pltpu
from jax.experimental.pallas import tpu_sc as plsc

L = 16
H = 128
W = 128
HW = H * W
NROW = 16 * 98
NWORK = 32
RPW = NROW // NWORK
TOPK = 9
NEG = -1e38
BIGI = 1 << 20
CPR = W // L


def _row_topk(row, rmbuf, cellbuf, candbuf, iota, lane_region):
    def _pa(r, m):
        vs = [row[r, pl.ds(u * L, L)] for u in range(CPR)]
        t0 = jnp.maximum(jnp.maximum(vs[0], vs[1]), jnp.maximum(vs[2], vs[3]))
        t1 = jnp.maximum(jnp.maximum(vs[4], vs[5]), jnp.maximum(vs[6], vs[7]))
        rm = jnp.maximum(t0, t1)
        rmbuf[r, :] = rm
        return jnp.maximum(m, rm)

    m = lax.fori_loop(0, H, _pa, jnp.full((L,), NEG, jnp.float32))
    msort = lax.sort(m)
    t = jnp.max(jnp.where(iota == L - TOPK, msort, NEG))

    cell_region = iota * H

    def _cc(i, carry):
        addr, rbase = carry
        rms = [rmbuf[i * 16 + u, :] for u in range(16)]
        sels = [rm >= t for rm in rms]
        incs = [jnp.where(s, 1, 0) for s in sels]
        for u in range(16):
            plsc.store_scatter(cellbuf, [addr], rbase + u * W, mask=sels[u])
            addr = addr + incs[u]
        return addr, rbase + 16 * W

    caddr, _ = lax.fori_loop(0, H // 16, _cc, (cell_region, iota))
    celloff = caddr - cell_region
    maxco = jnp.max(celloff)

    def _ex(r, addr):
        cb = plsc.load_gather(cellbuf, [cell_region + r])
        validc = celloff > r
        rr = (cb >> 7) & (H - 1)
        cc0 = cb & (W - 1)
        eidxs = [cb + s * L for s in range(CPR)]
        valss = [
            plsc.load_gather(row, [rr, cc0 + s * L]) for s in range(CPR)
        ]
        sels = [(v >= t) & validc for v in valss]
        incs = [jnp.where(s, 1, 0) for s in sels]
        for s in range(CPR):
            plsc.store_scatter(candbuf, [addr], eidxs[s], mask=sels[s])
            addr = addr + incs[s]
        return addr

    addr2 = lax.fori_loop(0, maxco, _ex, lane_region)
    off = addr2 - lane_region
    maxoff = jnp.max(off)

    def _gather_vals(r):
        cidx = plsc.load_gather(candbuf, [lane_region + r])
        valid = off > r
        v = plsc.load_gather(row, [(cidx >> 7) & (H - 1), cidx & (W - 1)])
        return cidx, jnp.where(valid, v, NEG)

    def _tm(k, carry):
        tval, tidx = carry
        cidx, vals = _gather_vals(k)
        sk, si = plsc.sort_key_val(vals, cidx, descending=True)
        keep = tval >= sk
        mval = jnp.where(keep, tval, sk)
        midx = jnp.where(keep, tidx, si)
        mk, mi = plsc.sort_key_val(mval, midx)
        return mk, mi

    cidx0, vals0 = _gather_vals(0)
    tval, tidx = plsc.sort_key_val(vals0, cidx0)
    tval, tidx = lax.fori_loop(1, maxoff, _tm, (tval, tidx))
    v1 = jnp.max(tval)
    v9 = jnp.max(jnp.where(iota == L - TOPK, tval, NEG))
    gt = tval > v9
    count_gt = jnp.sum(jnp.where(gt, 1, 0))
    need_eq = TOPK - count_gt

    def _em(k, e):
        cidx, vals = _gather_vals(k)
        eidx = jnp.where(vals == v9, cidx, BIGI)
        sdesc = lax.rev(lax.sort(eidx), (0,))
        return lax.sort(jnp.minimum(e, sdesc))

    e = lax.sort(jnp.where(vals0 == v9, cidx0, BIGI))
    e = lax.fori_loop(1, maxoff, _em, e)

    wg = jnp.where(gt, jnp.exp(tval - v1), 0.0)
    w9 = jnp.exp(jnp.broadcast_to(v9, (L,)) - jnp.broadcast_to(v1, (L,)))
    we = jnp.where(iota < need_eq, w9, 0.0)
    xg = (tidx & (W - 1)).astype(jnp.float32)
    yg = (tidx >> 7).astype(jnp.float32)
    xe = (e & (W - 1)).astype(jnp.float32)
    ye = (e >> 7).astype(jnp.float32)
    sw = wg + we
    sx = wg * xg + we * xe
    sy = wg * yg + we * ye
    tw = jnp.sum(sw)
    numer = jnp.where(iota == 0, jnp.sum(sx), jnp.sum(sy)) * 4.0
    denom = jnp.broadcast_to(tw, (L,))
    return numer / denom


def _body(
    in_hbm, out_hbm, rowa, rowb, rmbuf, cellbuf, candbuf, outbuf,
    sema, semb,
):
    cid = lax.axis_index("c")
    sid = lax.axis_index("s")
    wid = sid * 2 + cid
    n = wid // 2
    half = wid - n * 2
    c0 = half * RPW

    iota = lax.iota(jnp.int32, L)
    lane_region = iota * (HW // L)
    bufs = (rowa, rowb)
    sems = (sema, semb)

    pltpu.async_copy(in_hbm.at[n, 1, c0], rowa, sema)

    def _outer(k, _):
        for b in range(2):
            j = k * 2 + b

            @pl.when(j < RPW)
            def _():
                @pl.when(j + 1 < RPW)
                def _():
                    pltpu.async_copy(
                        in_hbm.at[n, 1, c0 + j + 1], bufs[1 - b], sems[1 - b]
                    )

                pltpu.make_async_copy(
                    in_hbm.at[n, 1, c0 + j], bufs[b], sems[b]
                ).wait()
                outv = _row_topk(
                    bufs[b], rmbuf, cellbuf, candbuf, iota, lane_region
                )
                rowi = jnp.broadcast_to((j * 2) >> 7, (L,))
                coli = ((j * 2) & (W - 1)) + iota
                plsc.store_scatter(outbuf, [rowi, coli], outv, mask=iota < 2)

        return 0

    lax.fori_loop(0, (RPW + 1) // 2, _outer, 0)
    pltpu.sync_copy(outbuf, out_hbm.at[wid])


@functools.partial(jax.jit, donate_argnums=())
def _run(x):
    mesh = plsc.VectorSubcoreMesh(core_axis_name="c", subcore_axis_name="s")
    kern = functools.partial(
        pl.kernel,
        mesh=mesh,
        compiler_params=pltpu.CompilerParams(
            needs_layout_passes=False, use_tc_tiling_on_sc=True
        ),
        out_type=jax.ShapeDtypeStruct((NWORK, 8, W), jnp.float32),
        scratch_types=[
            pltpu.VMEM((H, W), jnp.float32),
            pltpu.VMEM((H, W), jnp.float32),
            pltpu.VMEM((H, L), jnp.float32),
            pltpu.VMEM((H * L,), jnp.int32),
            pltpu.VMEM((HW,), jnp.int32),
            pltpu.VMEM((8, W), jnp.float32),
            pltpu.SemaphoreType.DMA,
            pltpu.SemaphoreType.DMA,
        ],
    )(_body)
    return kern(x)


def kernel(input):
    out = _run(input)
    return out.reshape(NWORK, 8 * W)[:, : 2 * RPW].reshape(16, 98, 2)


# speedup vs baseline: 1.0449x; 1.0449x over previous
import functools

import jax
import jax.numpy as jnp
from jax import lax
from jax.experimental import pallas as pl
from jax.experimental.pallas import tpu as pltpu
from jax.experimental.pallas import tpu_sc as plsc

L = 16
H = 128
W = 128
HW = H * W
NROW = 16 * 98
NWORK = 32
RPW = NROW // NWORK
TOPK = 9
NEG = -1e38
BIGI = 1 << 20
CPR = W // L


def _row_topk(row, rmbuf, cellbuf, candbuf, compact, iota, lane_region):
    def _pa(r, m):
        vs = [row[r, pl.ds(u * L, L)] for u in range(CPR)]
        t0 = jnp.maximum(jnp.maximum(vs[0], vs[1]), jnp.maximum(vs[2], vs[3]))
        t1 = jnp.maximum(jnp.maximum(vs[4], vs[5]), jnp.maximum(vs[6], vs[7]))
        rm = jnp.maximum(t0, t1)
        rmbuf[r, :] = rm
        return jnp.maximum(m, rm)

    m = lax.fori_loop(0, H, _pa, jnp.full((L,), NEG, jnp.float32))
    msort = lax.sort(m)
    t = jnp.max(jnp.where(iota == L - TOPK, msort, NEG))

    cell_region = iota * H

    def _cc(i, carry):
        addr, rbase = carry
        rms = [rmbuf[i * 16 + u, :] for u in range(16)]
        sels = [rm >= t for rm in rms]
        incs = [jnp.where(s, 1, 0) for s in sels]
        for u in range(16):
            plsc.store_scatter(cellbuf, [addr], rbase + u * W, mask=sels[u])
            addr = addr + incs[u]
        return addr, rbase + 16 * W

    caddr, _ = lax.fori_loop(0, H // 16, _cc, (cell_region, iota))
    celloff = caddr - cell_region
    ncell = jnp.sum(celloff)
    maxco = jnp.max(celloff)
    cexcl = plsc.cumsum(celloff) - celloff
    nck = (ncell + L - 1) // L

    def _pf1(k, _):
        compact[pl.ds(k * L, L)] = jnp.zeros((L,), jnp.int32)
        return 0

    lax.fori_loop(0, nck, _pf1, 0)

    def _pc1(r, _):
        sel = celloff > r
        vals = plsc.load_gather(cellbuf, [cell_region + r])
        plsc.store_scatter(compact, [cexcl + r], vals, mask=sel)
        return 0

    lax.fori_loop(0, maxco, _pc1, 0)

    def _ex(k, addr):
        cb = compact[pl.ds(k * L, L)]
        validc = (k * L + iota) < ncell
        rr = cb >> 7
        cc0 = cb & (W - 1)
        eidxs = [cb + s * L for s in range(CPR)]
        valss = [
            plsc.load_gather(row, [rr, cc0 + s * L]) for s in range(CPR)
        ]
        sels = [(v >= t) & validc for v in valss]
        incs = [jnp.where(s, 1, 0) for s in sels]
        for s in range(CPR):
            plsc.store_scatter(candbuf, [addr], eidxs[s], mask=sels[s])
            addr = addr + incs[s]
        return addr

    addr2 = lax.fori_loop(0, nck, _ex, lane_region)
    off = addr2 - lane_region
    maxoff = jnp.max(off)

    def _gather_vals(r):
        cidx = plsc.load_gather(candbuf, [lane_region + r])
        valid = off > r
        v = plsc.load_gather(row, [(cidx >> 7) & (H - 1), cidx & (W - 1)])
        return cidx, jnp.where(valid, v, NEG)

    def _tm(k, carry):
        tval, tidx = carry
        cidx, vals = _gather_vals(k)
        sk, si = plsc.sort_key_val(vals, cidx, descending=True)
        keep = tval >= sk
        mval = jnp.where(keep, tval, sk)
        midx = jnp.where(keep, tidx, si)
        mk, mi = plsc.sort_key_val(mval, midx)
        return mk, mi

    cidx0, vals0 = _gather_vals(0)
    tval, tidx = plsc.sort_key_val(vals0, cidx0)
    tval, tidx = lax.fori_loop(1, maxoff, _tm, (tval, tidx))
    v1 = jnp.max(tval)
    v9 = jnp.max(jnp.where(iota == L - TOPK, tval, NEG))
    gt = tval > v9
    count_gt = jnp.sum(jnp.where(gt, 1, 0))
    need_eq = TOPK - count_gt

    def _em(k, e):
        cidx, vals = _gather_vals(k)
        eidx = jnp.where(vals == v9, cidx, BIGI)
        sdesc = lax.rev(lax.sort(eidx), (0,))
        return lax.sort(jnp.minimum(e, sdesc))

    e = lax.sort(jnp.where(vals0 == v9, cidx0, BIGI))
    e = lax.fori_loop(1, maxoff, _em, e)

    wg = jnp.where(gt, jnp.exp(tval - v1), 0.0)
    w9 = jnp.exp(jnp.broadcast_to(v9, (L,)) - jnp.broadcast_to(v1, (L,)))
    we = jnp.where(iota < need_eq, w9, 0.0)
    xg = (tidx & (W - 1)).astype(jnp.float32)
    yg = (tidx >> 7).astype(jnp.float32)
    xe = (e & (W - 1)).astype(jnp.float32)
    ye = (e >> 7).astype(jnp.float32)
    sw = wg + we
    sx = wg * xg + we * xe
    sy = wg * yg + we * ye
    tw = jnp.sum(sw)
    numer = jnp.where(iota == 0, jnp.sum(sx), jnp.sum(sy)) * 4.0
    denom = jnp.broadcast_to(tw, (L,))
    return numer / denom


def _body(
    in_hbm, out_hbm, rowa, rowb, rmbuf, cellbuf, candbuf, compact, outbuf,
    sema, semb,
):
    cid = lax.axis_index("c")
    sid = lax.axis_index("s")
    wid = sid * 2 + cid
    n = wid // 2
    half = wid - n * 2
    c0 = half * RPW

    iota = lax.iota(jnp.int32, L)
    lane_region = iota * (HW // L)
    bufs = (rowa, rowb)
    sems = (sema, semb)

    pltpu.async_copy(in_hbm.at[n, 1, c0], rowa, sema)

    def _outer(k, _):
        for b in range(2):
            j = k * 2 + b

            @pl.when(j < RPW)
            def _():
                @pl.when(j + 1 < RPW)
                def _():
                    pltpu.async_copy(
                        in_hbm.at[n, 1, c0 + j + 1], bufs[1 - b], sems[1 - b]
                    )

                pltpu.make_async_copy(
                    in_hbm.at[n, 1, c0 + j], bufs[b], sems[b]
                ).wait()
                outv = _row_topk(
                    bufs[b], rmbuf, cellbuf, candbuf, compact, iota,
                    lane_region,
                )
                rowi = jnp.broadcast_to((j * 2) >> 7, (L,))
                coli = ((j * 2) & (W - 1)) + iota
                plsc.store_scatter(outbuf, [rowi, coli], outv, mask=iota < 2)

        return 0

    lax.fori_loop(0, (RPW + 1) // 2, _outer, 0)
    pltpu.sync_copy(outbuf, out_hbm.at[wid])


@functools.partial(jax.jit, donate_argnums=())
def _run(x):
    mesh = plsc.VectorSubcoreMesh(core_axis_name="c", subcore_axis_name="s")
    kern = functools.partial(
        pl.kernel,
        mesh=mesh,
        compiler_params=pltpu.CompilerParams(
            needs_layout_passes=False, use_tc_tiling_on_sc=True
        ),
        out_type=jax.ShapeDtypeStruct((NWORK, 8, W), jnp.float32),
        scratch_types=[
            pltpu.VMEM((H, W), jnp.float32),
            pltpu.VMEM((H, W), jnp.float32),
            pltpu.VMEM((H, L), jnp.float32),
            pltpu.VMEM((H * L,), jnp.int32),
            pltpu.VMEM((HW,), jnp.int32),
            pltpu.VMEM((HW,), jnp.int32),
            pltpu.VMEM((8, W), jnp.float32),
            pltpu.SemaphoreType.DMA,
            pltpu.SemaphoreType.DMA,
        ],
    )(_body)
    return kern(x)


def kernel(input):
    out = _run(input)
    return out.reshape(NWORK, 8 * W)[:, : 2 * RPW].reshape(16, 98, 2)
